# baseline (speedup 1.0000x reference)
import functools

import jax
import jax.numpy as jnp
from jax import lax
from jax.experimental import pallas as pl
from jax.experimental.pallas import tpu as pltpu
from jax.experimental.pallas import tpu_sc as plsc

N = 10000
E = 160000
K = 4
K3 = 64
RADIUS = 3.0

NC = 2
NS = 16
NW = NC * NS
LN = 16

CHN = 80
NCHUNK = 125
EB = 64
EP = 163840
EPW = EP // NW
BPW = EPW // EB
RW = 17 * EB
RPAD = RW + LN

COUT = [64, 64, 32, 32, 3]
COUT_PAD = [64, 64, 32, 32, 16]


def _mesh():
    return plsc.VectorSubcoreMesh(
        core_axis_name="c", subcore_axis_name="s", num_cores=NC, num_subcores=NS
    )

_SC_PARAMS = pltpu.CompilerParams(
    use_tc_tiling_on_sc=False, needs_layout_passes=False
)


def _sqrt16(q):
    qi = plsc.bitcast(q, jnp.int32)
    yi = lax.shift_right_logical(qi, 1) + 0x1FBD1DF5
    y = plsc.bitcast(yi, jnp.float32)
    for _ in range(3):
        y = 0.5 * (y + q / y)
    return y


def _geom_body(px_h, py_h, pz_h, src_h, dst_h, rec_h,
               px, py, pz, srcv, dstv, recbuf):
    wid = lax.axis_index("s") * NC + lax.axis_index("c")
    pltpu.sync_copy(px_h, px)
    pltpu.sync_copy(py_h, py)
    pltpu.sync_copy(pz_h, pz)
    ebase = wid * EPW
    pltpu.sync_copy(src_h.at[pl.ds(ebase, EPW)], srcv)
    pltpu.sync_copy(dst_h.at[pl.ds(ebase, EPW)], dstv)

    scale = 2.0 / RADIUS
    lane_ids = lax.iota(jnp.int32, LN)

    for half in range(2):
        def body(i, carry, half=half):
            off = half * (EPW // 2) + i * LN
            bloc = i // 4
            lane = (i % 4) * LN
            rbase = bloc * RW
            s = srcv[pl.ds(off, LN)]
            d = dstv[pl.ds(off, LN)]
            rx = (plsc.load_gather(px, [s]) - plsc.load_gather(px, [d])) * scale
            ry = (plsc.load_gather(py, [s]) - plsc.load_gather(py, [d])) * scale
            rz = (plsc.load_gather(pz, [s]) - plsc.load_gather(pz, [d])) * scale
            s2 = rx * rx + ry * ry + rz * rz + 1e-12
            linf = jnp.maximum(jnp.maximum(jnp.abs(rx), jnp.abs(ry)), jnp.abs(rz))
            linf = jnp.maximum(linf, 1e-8)
            ratio = _sqrt16(s2 / (linf * linf))
            gx = (jnp.clip(rx * ratio, -1.0, 1.0) + 1.0) * (0.5 * (K - 1))
            gy = (jnp.clip(ry * ratio, -1.0, 1.0) + 1.0) * (0.5 * (K - 1))
            gz = (jnp.clip(rz * ratio, -1.0, 1.0) + 1.0) * (0.5 * (K - 1))
            g0x = jnp.clip(gx.astype(jnp.int32), 0, K - 2)
            g0y = jnp.clip(gy.astype(jnp.int32), 0, K - 2)
            g0z = jnp.clip(gz.astype(jnp.int32), 0, K - 2)
            fx = gx - g0x.astype(jnp.float32)
            fy = gy - g0y.astype(jnp.float32)
            fz = gz - g0z.astype(jnp.float32)
            wx = (1.0 - fx, fx)
            wy = (1.0 - fy, fy)
            wz = (1.0 - fz, fz)
            base = (g0x * K + g0y) * K + g0z + s * K3
            widx = rbase + 8 * EB + (lane + lane_ids) * 8
            kidx = 0
            for dx in (0, 1):
                for dy in (0, 1):
                    wxy = wx[dx] * wy[dy]
                    for dz in (0, 1):
                        recbuf[pl.ds(rbase + kidx * EB + lane, LN)] = (
                            base + (dx * 16 + dy * 4 + dz)
                        )
                        plsc.store_scatter(
                            recbuf, [widx + kidx],
                            plsc.bitcast(wxy * wz[dz], jnp.int32),
                        )
                        kidx += 1
            recbuf[pl.ds(rbase + 16 * EB + lane, LN)] = d
            return carry

        lax.fori_loop(0, (EPW // 2) // LN, body, 0)
        pltpu.sync_copy(
            recbuf,
            rec_h.at[pl.ds((wid * BPW + half * (BPW // 2)) * RW, (BPW // 2) * RW)],
        )


_geom = functools.partial(
    pl.kernel,
    out_type=jax.ShapeDtypeStruct((EP // EB * RW,), jnp.int32),
    mesh=_mesh(),
    compiler_params=_SC_PARAMS,
    scratch_types=[
        pltpu.VMEM((N,), jnp.float32),
        pltpu.VMEM((N,), jnp.float32),
        pltpu.VMEM((N,), jnp.float32),
        pltpu.VMEM((EPW,), jnp.int32),
        pltpu.VMEM((EPW,), jnp.int32),
        pltpu.VMEM(((BPW // 2) * RW,), jnp.int32),
    ],
)(_geom_body)


def _make_conv(C, packed):
    CW = C // 2 if packed else C

    def body(t_h, rec_h, rp_h, b_h, y_h,
             rp_v, b_v, yl, rec_v, dl_v, rows, semi, semg):
        wid = lax.axis_index("s") * NC + lax.axis_index("c")
        pltpu.sync_copy(rp_h, rp_v)
        pltpu.sync_copy(b_h, b_v)

        def fire_rec(blk, par):
            pltpu.async_copy(
                rec_h.at[pl.ds(blk * RW, RW)],
                rec_v.at[par, pl.ds(0, RW)],
                semi.at[par],
            )

        def wait_rec(par):
            pltpu.make_async_copy(
                rec_h.at[pl.ds(0, RW)],
                rec_v.at[par, pl.ds(0, RW)],
                semi.at[par],
            ).wait()

        def fire_gathers(par):
            for k in range(8):
                pltpu.async_copy(
                    t_h.at[rec_v.at[par, pl.ds(k * EB, EB)]],
                    rows.at[par, k],
                    semg.at[par],
                )

        def wait_gathers(par):
            for k in range(8):
                pltpu.make_async_copy(
                    t_h.at[pl.ds(0, EB)], rows.at[par, k], semg.at[par]
                ).wait()

        def chunk_body(ci, carry):
            c = wid + ci * NW

            @pl.when(c < NCHUNK)
            def _():
                n0 = c * CHN
                ev = rp_v[pl.ds(c, LN)]
                e0 = ev[0]
                e1 = ev[1]
                b0 = e0 // EB
                nb = (e1 + EB - 1) // EB - b0

                def initb(r, cr):
                    for j in range(C // LN):
                        yl[pl.ds(r * C + j * LN, LN)] = b_v[pl.ds(j * LN, LN)]
                    return cr

                lax.fori_loop(0, CHN + 1, initb, 0)

                @pl.when(nb > 0)
                def _():
                    fire_rec(b0, 0)
                    wait_rec(0)
                    fire_gathers(0)

                    @pl.when(nb > 1)
                    def _():
                        fire_rec(b0 + 1, 1)

                def blk(b, cr):
                    par = b % 2
                    wait_gathers(par)
                    for j in range(EB // LN):
                        eg = lax.iota(jnp.int32, LN) + ((b0 + b) * EB + j * LN)
                        val = (eg >= e0) & (eg < e1)
                        dvec = rec_v[par, pl.ds(16 * EB + j * LN, LN)]
                        dl_v[par, pl.ds(j * LN, LN)] = jnp.where(
                            val, dvec - n0, CHN
                        )

                    @pl.when(b + 1 < nb)
                    def _():
                        wait_rec(1 - par)
                        fire_gathers(1 - par)

                    def edge(ei, cr2):
                        dloc = dl_v[par, pl.ds(ei, LN)][0]
                        wvec = plsc.bitcast(
                            rec_v[par, pl.ds(8 * EB + ei * 8, LN)], jnp.float32
                        )
                        rbase = dloc * C
                        if packed:
                            for j in range(C // 32):
                                acc_a = None
                                acc_b = None
                                for k in range(8):
                                    v = rows[par, k, ei, pl.ds(j * LN, LN)]
                                    va = plsc.bitcast(
                                        lax.shift_left(v, 16), jnp.float32)
                                    vb = plsc.bitcast(
                                        v & jnp.int32(-65536), jnp.float32)
                                    if k == 0:
                                        acc_a = wvec[0] * va
                                        acc_b = wvec[0] * vb
                                    else:
                                        acc_a += wvec[k] * va
                                        acc_b += wvec[k] * vb
                                plsc.addupdate(
                                    yl.at[pl.ds(rbase + j * 32, LN)], acc_a)
                                plsc.addupdate(
                                    yl.at[pl.ds(rbase + j * 32 + LN, LN)], acc_b)
                        else:
                            for j in range(C // LN):
                                acc = wvec[0] * rows[par, 0, ei, pl.ds(j * LN, LN)]
                                for k in range(1, 8):
                                    acc += wvec[k] * rows[par, k, ei,
                                                          pl.ds(j * LN, LN)]
                                plsc.addupdate(yl.at[pl.ds(rbase + j * LN, LN)],
                                               acc)
                        return cr2

                    lax.fori_loop(0, EB, edge, 0)

                    @pl.when(b + 2 < nb)
                    def _():
                        fire_rec(b0 + b + 2, par)

                    return cr

                lax.fori_loop(0, nb, blk, 0)
                pltpu.sync_copy(yl.at[pl.ds(0, CHN * C)],
                                y_h.at[pl.ds(n0 * C, CHN * C)])

            return carry

        lax.fori_loop(0, (NCHUNK + NW - 1) // NW, chunk_body, 0)

    return functools.partial(
        pl.kernel,
        out_type=jax.ShapeDtypeStruct((N * C,), jnp.float32),
        mesh=_mesh(),
        compiler_params=_SC_PARAMS,
        scratch_types=[
            pltpu.VMEM((144,), jnp.int32),
            pltpu.VMEM((C,), jnp.float32),
            pltpu.VMEM(((CHN + 1) * C,), jnp.float32),
            pltpu.VMEM((2, RPAD), jnp.int32),
            pltpu.VMEM((2, EB + LN), jnp.int32),
            pltpu.VMEM((2, 8, EB, CW),
                       jnp.int32 if packed else jnp.float32),
            pltpu.SemaphoreType.DMA((2,)),
            pltpu.SemaphoreType.DMA((2,)),
        ],
    )(body)


_PACKED = [True, True, True, True, False]
_CONVS = [_make_conv(c, p) for c, p in zip(COUT_PAD, _PACKED)]


def _perm(c):
    import numpy as _np
    out = []
    for g in range(c // 32):
        out.extend(range(g * 32, (g + 1) * 32, 2))
        out.extend(range(g * 32 + 1, (g + 1) * 32, 2))
    return _np.asarray(out, dtype=_np.int32)


def _mm(x, w2, relu, bf16_out):
    KK = w2.shape[0]
    CC = w2.shape[1]
    BN, BC = 400, 512
    odt = jnp.bfloat16 if bf16_out else jnp.float32

    def body(x_ref, w_ref, o_ref):
        xb = x_ref[:]
        if relu:
            xb = jnp.maximum(xb, 0.0)
        o_ref[:] = jnp.dot(
            xb, w_ref[:], preferred_element_type=jnp.float32
        ).astype(odt)

    return pl.pallas_call(
        body,
        grid=(N // BN, CC // BC),
        in_specs=[
            pl.BlockSpec((BN, KK), lambda i, j: (i, 0)),
            pl.BlockSpec((KK, BC), lambda i, j: (0, j)),
        ],
        out_specs=pl.BlockSpec((BN, BC), lambda i, j: (i, j)),
        out_shape=jax.ShapeDtypeStruct((N, CC), odt),
    )(x, w2)


def kernel(feats, pos, edge_index, W0, b0, W1, b1, W2, b2, W3, b3, W4, b4):
    src = edge_index[0]
    dst = edge_index[1]
    order = jnp.argsort(dst)
    src_s = src[order].astype(jnp.int32)
    dst_s = dst[order].astype(jnp.int32)
    rowptr = jnp.searchsorted(
        dst_s, jnp.arange(NCHUNK + 1, dtype=jnp.int32) * CHN
    ).astype(jnp.int32)
    rowptr = jnp.pad(rowptr, (0, 144 - (NCHUNK + 1)))
    srcp = jnp.pad(src_s, (0, EP - E))
    dstp = jnp.pad(dst_s, (0, EP - E))
    px = jnp.asarray(pos[:, 0])
    py = jnp.asarray(pos[:, 1])
    pz = jnp.asarray(pos[:, 2])

    rec = _geom(px, py, pz, srcp, dstp)

    params = [(W0, b0), (W1, b1), (W2, b2), (W3, b3), (W4, b4)]
    x = jnp.pad(feats, ((0, 0), (0, 8 - feats.shape[1])))
    y = None
    prev_perm = None
    for i, (W, b) in enumerate(params):
        cin = W.shape[1]
        kk = x.shape[1]
        cout = W.shape[2]
        cpad = COUT_PAD[i]
        w2 = jnp.transpose(W, (1, 0, 2))
        w2 = jnp.pad(w2, ((0, kk - cin), (0, 0), (0, cpad - cout)))
        w2 = w2.reshape(kk, K3 * cpad)
        if prev_perm is not None:
            w2 = w2[prev_perm, :]
        bp = jnp.pad(b, (0, cpad - cout))
        if _PACKED[i]:
            bp = bp[_perm(cpad)]
        T = _mm(x, w2, relu=(i > 0), bf16_out=_PACKED[i])
        if _PACKED[i]:
            T2 = lax.bitcast_convert_type(
                T.reshape(N * K3, cpad // 2, 2), jnp.int32)
        else:
            T2 = T.reshape(N * K3, cpad)
        y = _CONVS[i](T2, rec, rowptr, bp)
        prev_perm = _perm(cpad) if _PACKED[i] else None
        if i < len(params) - 1:
            x = y.reshape(N, cpad)
    return y.reshape(N, COUT_PAD[-1])[:, : COUT[-1]]


# speedup vs baseline: 34.5671x; 34.5671x over previous
import functools

import jax
import jax.numpy as jnp
from jax import lax
from jax.experimental import pallas as pl
from jax.experimental.pallas import tpu as pltpu
from jax.experimental.pallas import tpu_sc as plsc

N = 10000
E = 160000
K = 4
K3 = 64
RADIUS = 3.0

NC = 2
NS = 16
NW = NC * NS
LN = 16

CHN = 80
NCHUNK = 125
EB = 64
EP = 163840
EPW = EP // NW
BPW = EPW // EB
RW = 17 * EB
RPAD = RW + LN

COUT = [64, 64, 32, 32, 3]
COUT_PAD = [64, 64, 32, 32, 16]


def _mesh():
    return plsc.VectorSubcoreMesh(
        core_axis_name="c", subcore_axis_name="s", num_cores=NC, num_subcores=NS
    )

_SC_PARAMS = pltpu.CompilerParams(
    use_tc_tiling_on_sc=False, needs_layout_passes=False
)


def _sqrt16(q):
    qi = plsc.bitcast(q, jnp.int32)
    yi = lax.shift_right_logical(qi, 1) + 0x1FBD1DF5
    y = plsc.bitcast(yi, jnp.float32)
    for _ in range(3):
        y = 0.5 * (y + q / y)
    return y


def _geom_body(px_h, py_h, pz_h, src_h, dst_h, rec_h,
               px, py, pz, srcv, dstv, recbuf):
    wid = lax.axis_index("s") * NC + lax.axis_index("c")
    pltpu.sync_copy(px_h, px)
    pltpu.sync_copy(py_h, py)
    pltpu.sync_copy(pz_h, pz)
    ebase = wid * EPW
    pltpu.sync_copy(src_h.at[pl.ds(ebase, EPW)], srcv)
    pltpu.sync_copy(dst_h.at[pl.ds(ebase, EPW)], dstv)

    scale = 2.0 / RADIUS
    lane_ids = lax.iota(jnp.int32, LN)

    for half in range(2):
        def body(i, carry, half=half):
            off = half * (EPW // 2) + i * LN
            bloc = i // 4
            lane = (i % 4) * LN
            rbase = bloc * RW
            s = srcv[pl.ds(off, LN)]
            d = dstv[pl.ds(off, LN)]
            rx = (plsc.load_gather(px, [s]) - plsc.load_gather(px, [d])) * scale
            ry = (plsc.load_gather(py, [s]) - plsc.load_gather(py, [d])) * scale
            rz = (plsc.load_gather(pz, [s]) - plsc.load_gather(pz, [d])) * scale
            s2 = rx * rx + ry * ry + rz * rz + 1e-12
            linf = jnp.maximum(jnp.maximum(jnp.abs(rx), jnp.abs(ry)), jnp.abs(rz))
            linf = jnp.maximum(linf, 1e-8)
            ratio = _sqrt16(s2 / (linf * linf))
            gx = (jnp.clip(rx * ratio, -1.0, 1.0) + 1.0) * (0.5 * (K - 1))
            gy = (jnp.clip(ry * ratio, -1.0, 1.0) + 1.0) * (0.5 * (K - 1))
            gz = (jnp.clip(rz * ratio, -1.0, 1.0) + 1.0) * (0.5 * (K - 1))
            g0x = jnp.clip(gx.astype(jnp.int32), 0, K - 2)
            g0y = jnp.clip(gy.astype(jnp.int32), 0, K - 2)
            g0z = jnp.clip(gz.astype(jnp.int32), 0, K - 2)
            fx = gx - g0x.astype(jnp.float32)
            fy = gy - g0y.astype(jnp.float32)
            fz = gz - g0z.astype(jnp.float32)
            wx = (1.0 - fx, fx)
            wy = (1.0 - fy, fy)
            wz = (1.0 - fz, fz)
            base = (g0x * K + g0y) * K + g0z + s * K3
            widx = rbase + 8 * EB + (lane + lane_ids) * 8
            kidx = 0
            for dx in (0, 1):
                for dy in (0, 1):
                    wxy = wx[dx] * wy[dy]
                    for dz in (0, 1):
                        recbuf[pl.ds(rbase + kidx * EB + lane, LN)] = (
                            base + (dx * 16 + dy * 4 + dz)
                        )
                        plsc.store_scatter(
                            recbuf, [widx + kidx],
                            plsc.bitcast(wxy * wz[dz], jnp.int32),
                        )
                        kidx += 1
            recbuf[pl.ds(rbase + 16 * EB + lane, LN)] = d
            return carry

        lax.fori_loop(0, (EPW // 2) // LN, body, 0)
        pltpu.sync_copy(
            recbuf,
            rec_h.at[pl.ds((wid * BPW + half * (BPW // 2)) * RW, (BPW // 2) * RW)],
        )


_geom = functools.partial(
    pl.kernel,
    out_type=jax.ShapeDtypeStruct((EP // EB * RW,), jnp.int32),
    mesh=_mesh(),
    compiler_params=_SC_PARAMS,
    scratch_types=[
        pltpu.VMEM((N,), jnp.float32),
        pltpu.VMEM((N,), jnp.float32),
        pltpu.VMEM((N,), jnp.float32),
        pltpu.VMEM((EPW,), jnp.int32),
        pltpu.VMEM((EPW,), jnp.int32),
        pltpu.VMEM(((BPW // 2) * RW,), jnp.int32),
    ],
)(_geom_body)


def _make_conv(C, packed):
    CW = C // 2 if packed else C

    def body(t_h, rec_h, rp_h, b_h, y_h,
             rp_v, b_v, yl, rec_v, dl_v, rows, semi, semg):
        wid = lax.axis_index("s") * NC + lax.axis_index("c")
        pltpu.sync_copy(rp_h, rp_v)
        pltpu.sync_copy(b_h, b_v)

        def fire_rec(blk, par):
            pltpu.async_copy(
                rec_h.at[pl.ds(blk * RW, RW)],
                rec_v.at[par, pl.ds(0, RW)],
                semi.at[par],
            )

        def wait_rec(par):
            pltpu.make_async_copy(
                rec_h.at[pl.ds(0, RW)],
                rec_v.at[par, pl.ds(0, RW)],
                semi.at[par],
            ).wait()

        def fire_gathers(par):
            for k in range(8):
                pltpu.async_copy(
                    t_h.at[rec_v.at[par, pl.ds(k * EB, EB)]],
                    rows.at[par, k],
                    semg.at[par],
                )

        def wait_gathers(par):
            for k in range(8):
                pltpu.make_async_copy(
                    t_h.at[pl.ds(0, EB)], rows.at[par, k], semg.at[par]
                ).wait()

        def chunk_body(ci, carry):
            c = wid + ci * NW

            @pl.when(c < NCHUNK)
            def _():
                n0 = c * CHN
                ev = rp_v[pl.ds(c, LN)]
                e0 = ev[0]
                e1 = ev[1]
                b0 = e0 // EB
                nb = (e1 + EB - 1) // EB - b0

                def initb(r, cr):
                    for j in range(C // LN):
                        yl[pl.ds(r * C + j * LN, LN)] = b_v[pl.ds(j * LN, LN)]
                    return cr

                lax.fori_loop(0, CHN + 1, initb, 0)

                @pl.when(nb > 0)
                def _():
                    fire_rec(b0, 0)
                    wait_rec(0)
                    fire_gathers(0)

                    @pl.when(nb > 1)
                    def _():
                        fire_rec(b0 + 1, 1)

                def blk(b, cr):
                    par = b % 2
                    wait_gathers(par)
                    for j in range(EB // LN):
                        eg = lax.iota(jnp.int32, LN) + ((b0 + b) * EB + j * LN)
                        val = (eg >= e0) & (eg < e1)
                        dvec = rec_v[par, pl.ds(16 * EB + j * LN, LN)]
                        dl_v[par, pl.ds(j * LN, LN)] = jnp.where(
                            val, dvec - n0, CHN
                        )

                    @pl.when(b + 1 < nb)
                    def _():
                        wait_rec(1 - par)
                        fire_gathers(1 - par)

                    def edge(ei, cr2):
                        dloc = dl_v[par, pl.ds(ei, LN)][0]
                        wvec = plsc.bitcast(
                            rec_v[par, pl.ds(8 * EB + ei * 8, LN)], jnp.float32
                        )
                        rbase = dloc * C
                        if packed:
                            for j in range(C // 32):
                                acc_a = None
                                acc_b = None
                                for k in range(8):
                                    v = plsc.bitcast(
                                        rows[par, k, ei, pl.ds(j * 32, 32)],
                                        jnp.int32)
                                    va = plsc.bitcast(
                                        lax.shift_left(v, 16), jnp.float32)
                                    vb = plsc.bitcast(
                                        v & jnp.int32(-65536), jnp.float32)
                                    if k == 0:
                                        acc_a = wvec[0] * va
                                        acc_b = wvec[0] * vb
                                    else:
                                        acc_a += wvec[k] * va
                                        acc_b += wvec[k] * vb
                                plsc.addupdate(
                                    yl.at[pl.ds(rbase + j * 32, LN)], acc_a)
                                plsc.addupdate(
                                    yl.at[pl.ds(rbase + j * 32 + LN, LN)], acc_b)
                        else:
                            for j in range(C // LN):
                                acc = wvec[0] * rows[par, 0, ei, pl.ds(j * LN, LN)]
                                for k in range(1, 8):
                                    acc += wvec[k] * rows[par, k, ei,
                                                          pl.ds(j * LN, LN)]
                                plsc.addupdate(yl.at[pl.ds(rbase + j * LN, LN)],
                                               acc)
                        return cr2

                    lax.fori_loop(0, EB, edge, 0)

                    @pl.when(b + 2 < nb)
                    def _():
                        fire_rec(b0 + b + 2, par)

                    return cr

                lax.fori_loop(0, nb, blk, 0)
                pltpu.sync_copy(yl.at[pl.ds(0, CHN * C)],
                                y_h.at[pl.ds(n0 * C, CHN * C)])

            return carry

        lax.fori_loop(0, (NCHUNK + NW - 1) // NW, chunk_body, 0)

    return functools.partial(
        pl.kernel,
        out_type=jax.ShapeDtypeStruct((N * C,), jnp.float32),
        mesh=_mesh(),
        compiler_params=_SC_PARAMS,
        scratch_types=[
            pltpu.VMEM((144,), jnp.int32),
            pltpu.VMEM((C,), jnp.float32),
            pltpu.VMEM(((CHN + 1) * C,), jnp.float32),
            pltpu.VMEM((2, RPAD), jnp.int32),
            pltpu.VMEM((2, EB + LN), jnp.int32),
            pltpu.VMEM((2, 8, EB, C),
                       jnp.bfloat16 if packed else jnp.float32),
            pltpu.SemaphoreType.DMA((2,)),
            pltpu.SemaphoreType.DMA((2,)),
        ],
    )(body)


_PACKED = [True, True, True, True, False]
_CONVS = [_make_conv(c, p) for c, p in zip(COUT_PAD, _PACKED)]


def _perm(c):
    import numpy as _np
    out = []
    for g in range(c // 32):
        out.extend(range(g * 32, (g + 1) * 32, 2))
        out.extend(range(g * 32 + 1, (g + 1) * 32, 2))
    return _np.asarray(out, dtype=_np.int32)


def _mm(x, w2, relu, bf16_out):
    KK = w2.shape[0]
    CC = w2.shape[1]
    BN, BC = 400, 512
    odt = jnp.bfloat16 if bf16_out else jnp.float32

    def body(x_ref, w_ref, o_ref):
        xb = x_ref[:]
        if relu:
            xb = jnp.maximum(xb, 0.0)
        o_ref[:] = jnp.dot(
            xb, w_ref[:], preferred_element_type=jnp.float32
        ).astype(odt)

    return pl.pallas_call(
        body,
        grid=(N // BN, CC // BC),
        in_specs=[
            pl.BlockSpec((BN, KK), lambda i, j: (i, 0)),
            pl.BlockSpec((KK, BC), lambda i, j: (0, j)),
        ],
        out_specs=pl.BlockSpec((BN, BC), lambda i, j: (i, j)),
        out_shape=jax.ShapeDtypeStruct((N, CC), odt),
    )(x, w2)


def kernel(feats, pos, edge_index, W0, b0, W1, b1, W2, b2, W3, b3, W4, b4):
    src = edge_index[0]
    dst = edge_index[1]
    order = jnp.argsort(dst)
    src_s = src[order].astype(jnp.int32)
    dst_s = dst[order].astype(jnp.int32)
    rowptr = jnp.searchsorted(
        dst_s, jnp.arange(NCHUNK + 1, dtype=jnp.int32) * CHN
    ).astype(jnp.int32)
    rowptr = jnp.pad(rowptr, (0, 144 - (NCHUNK + 1)))
    srcp = jnp.pad(src_s, (0, EP - E))
    dstp = jnp.pad(dst_s, (0, EP - E))
    px = jnp.asarray(pos[:, 0])
    py = jnp.asarray(pos[:, 1])
    pz = jnp.asarray(pos[:, 2])

    rec = _geom(px, py, pz, srcp, dstp)

    params = [(W0, b0), (W1, b1), (W2, b2), (W3, b3), (W4, b4)]
    x = jnp.pad(feats, ((0, 0), (0, 8 - feats.shape[1])))
    y = None
    prev_perm = None
    for i, (W, b) in enumerate(params):
        cin = W.shape[1]
        kk = x.shape[1]
        cout = W.shape[2]
        cpad = COUT_PAD[i]
        w2 = jnp.transpose(W, (1, 0, 2))
        w2 = jnp.pad(w2, ((0, kk - cin), (0, 0), (0, cpad - cout)))
        w2 = w2.reshape(kk, K3 * cpad)
        if prev_perm is not None:
            w2 = w2[prev_perm, :]
        bp = jnp.pad(b, (0, cpad - cout))
        if _PACKED[i]:
            bp = bp[_perm(cpad)]
        T = _mm(x, w2, relu=(i > 0), bf16_out=_PACKED[i])
        T2 = T.reshape(N * K3, cpad)
        y = _CONVS[i](T2, rec, rowptr, bp)
        prev_perm = _perm(cpad) if _PACKED[i] else None
        if i < len(params) - 1:
            x = y.reshape(N, cpad)
    return y.reshape(N, COUT_PAD[-1])[:, : COUT[-1]]


# speedup vs baseline: 34.9700x; 1.0117x over previous
import functools

import jax
import jax.numpy as jnp
from jax import lax
from jax.experimental import pallas as pl
from jax.experimental.pallas import tpu as pltpu
from jax.experimental.pallas import tpu_sc as plsc

N = 10000
E = 160000
K = 4
K3 = 64
RADIUS = 3.0

NC = 2
NS = 16
NW = NC * NS
LN = 16

CHN = 80
NCHUNK = 125
EB = 64
EP = 163840
EPW = EP // NW
BPW = EPW // EB
RW = 17 * EB
RPAD = RW + LN

COUT = [64, 64, 32, 32, 3]
COUT_PAD = [64, 64, 32, 32, 16]


def _mesh():
    return plsc.VectorSubcoreMesh(
        core_axis_name="c", subcore_axis_name="s", num_cores=NC, num_subcores=NS
    )

_SC_PARAMS = pltpu.CompilerParams(
    use_tc_tiling_on_sc=False, needs_layout_passes=False
)


def _sqrt16(q):
    qi = plsc.bitcast(q, jnp.int32)
    yi = lax.shift_right_logical(qi, 1) + 0x1FBD1DF5
    y = plsc.bitcast(yi, jnp.float32)
    for _ in range(3):
        y = 0.5 * (y + q / y)
    return y


def _geom_body(px_h, py_h, pz_h, src_h, dst_h, rec_h,
               px, py, pz, srcv, dstv, recbuf):
    wid = lax.axis_index("s") * NC + lax.axis_index("c")
    pltpu.sync_copy(px_h, px)
    pltpu.sync_copy(py_h, py)
    pltpu.sync_copy(pz_h, pz)
    ebase = wid * EPW
    pltpu.sync_copy(src_h.at[pl.ds(ebase, EPW)], srcv)
    pltpu.sync_copy(dst_h.at[pl.ds(ebase, EPW)], dstv)

    scale = 2.0 / RADIUS
    lane_ids = lax.iota(jnp.int32, LN)

    for half in range(2):
        def body(i, carry, half=half):
            off = half * (EPW // 2) + i * LN
            bloc = i // 4
            lane = (i % 4) * LN
            rbase = bloc * RW
            s = srcv[pl.ds(off, LN)]
            d = dstv[pl.ds(off, LN)]
            rx = (plsc.load_gather(px, [s]) - plsc.load_gather(px, [d])) * scale
            ry = (plsc.load_gather(py, [s]) - plsc.load_gather(py, [d])) * scale
            rz = (plsc.load_gather(pz, [s]) - plsc.load_gather(pz, [d])) * scale
            s2 = rx * rx + ry * ry + rz * rz + 1e-12
            linf = jnp.maximum(jnp.maximum(jnp.abs(rx), jnp.abs(ry)), jnp.abs(rz))
            linf = jnp.maximum(linf, 1e-8)
            ratio = _sqrt16(s2 / (linf * linf))
            gx = (jnp.clip(rx * ratio, -1.0, 1.0) + 1.0) * (0.5 * (K - 1))
            gy = (jnp.clip(ry * ratio, -1.0, 1.0) + 1.0) * (0.5 * (K - 1))
            gz = (jnp.clip(rz * ratio, -1.0, 1.0) + 1.0) * (0.5 * (K - 1))
            g0x = jnp.clip(gx.astype(jnp.int32), 0, K - 2)
            g0y = jnp.clip(gy.astype(jnp.int32), 0, K - 2)
            g0z = jnp.clip(gz.astype(jnp.int32), 0, K - 2)
            fx = gx - g0x.astype(jnp.float32)
            fy = gy - g0y.astype(jnp.float32)
            fz = gz - g0z.astype(jnp.float32)
            wx = (1.0 - fx, fx)
            wy = (1.0 - fy, fy)
            wz = (1.0 - fz, fz)
            base = (g0x * K + g0y) * K + g0z + s * K3
            widx = rbase + 8 * EB + (lane + lane_ids) * 8
            kidx = 0
            for dx in (0, 1):
                for dy in (0, 1):
                    wxy = wx[dx] * wy[dy]
                    for dz in (0, 1):
                        recbuf[pl.ds(rbase + kidx * EB + lane, LN)] = (
                            base + (dx * 16 + dy * 4 + dz)
                        )
                        plsc.store_scatter(
                            recbuf, [widx + kidx],
                            plsc.bitcast(wxy * wz[dz], jnp.int32),
                        )
                        kidx += 1
            recbuf[pl.ds(rbase + 16 * EB + lane, LN)] = d
            return carry

        lax.fori_loop(0, (EPW // 2) // LN, body, 0)
        pltpu.sync_copy(
            recbuf,
            rec_h.at[pl.ds((wid * BPW + half * (BPW // 2)) * RW, (BPW // 2) * RW)],
        )


_geom = functools.partial(
    pl.kernel,
    out_type=jax.ShapeDtypeStruct((EP // EB * RW,), jnp.int32),
    mesh=_mesh(),
    compiler_params=_SC_PARAMS,
    scratch_types=[
        pltpu.VMEM((N,), jnp.float32),
        pltpu.VMEM((N,), jnp.float32),
        pltpu.VMEM((N,), jnp.float32),
        pltpu.VMEM((EPW,), jnp.int32),
        pltpu.VMEM((EPW,), jnp.int32),
        pltpu.VMEM(((BPW // 2) * RW,), jnp.int32),
    ],
)(_geom_body)


def _make_conv(C, packed):
    CW = C // 2 if packed else C

    def body(t_h, rec_h, rp_h, b_h, y_h,
             rp_v, b_v, yl, rec_v, dl_v, rows, semi, semg):
        wid = lax.axis_index("s") * NC + lax.axis_index("c")
        pltpu.sync_copy(rp_h, rp_v)
        pltpu.sync_copy(b_h, b_v)

        def fire_rec(blk, par):
            pltpu.async_copy(
                rec_h.at[pl.ds(blk * RW, RW)],
                rec_v.at[par, pl.ds(0, RW)],
                semi.at[par],
            )

        def wait_rec(par):
            pltpu.make_async_copy(
                rec_h.at[pl.ds(0, RW)],
                rec_v.at[par, pl.ds(0, RW)],
                semi.at[par],
            ).wait()

        def fire_gathers(par):
            for k in range(8):
                pltpu.async_copy(
                    t_h.at[rec_v.at[par, pl.ds(k * EB, EB)]],
                    rows.at[par, k],
                    semg.at[par],
                )

        def wait_gathers(par):
            for k in range(8):
                pltpu.make_async_copy(
                    t_h.at[pl.ds(0, EB)], rows.at[par, k], semg.at[par]
                ).wait()

        def chunk_body(ci, carry):
            c = wid + ci * NW

            @pl.when(c < NCHUNK)
            def _():
                n0 = c * CHN
                ev = rp_v[pl.ds(c, LN)]
                e0 = ev[0]
                e1 = ev[1]
                b0 = e0 // EB
                nb = (e1 + EB - 1) // EB - b0

                def initb(r, cr):
                    for j in range(C // LN):
                        yl[pl.ds(r * C + j * LN, LN)] = b_v[pl.ds(j * LN, LN)]
                    return cr

                lax.fori_loop(0, CHN + 1, initb, 0)

                @pl.when(nb > 0)
                def _():
                    fire_rec(b0, 0)
                    wait_rec(0)
                    fire_gathers(0)

                    @pl.when(nb > 1)
                    def _():
                        fire_rec(b0 + 1, 1)

                def blk(b, cr):
                    par = b % 2
                    wait_gathers(par)
                    for j in range(EB // LN):
                        eg = lax.iota(jnp.int32, LN) + ((b0 + b) * EB + j * LN)
                        val = (eg >= e0) & (eg < e1)
                        dvec = rec_v[par, pl.ds(16 * EB + j * LN, LN)]
                        dl_v[par, pl.ds(j * LN, LN)] = jnp.where(
                            val, dvec - n0, CHN
                        )

                    @pl.when(b + 1 < nb)
                    def _():
                        wait_rec(1 - par)
                        fire_gathers(1 - par)

                    def edge(ei, cr2):
                        dloc = dl_v[par, pl.ds(ei, LN)][0]
                        wvec = plsc.bitcast(
                            rec_v[par, pl.ds(8 * EB + ei * 8, LN)], jnp.float32
                        )
                        rbase = dloc * C
                        if packed:
                            for j in range(C // 32):
                                acc_a = None
                                acc_b = None
                                for k in range(8):
                                    v = plsc.bitcast(
                                        rows[par, k, ei, pl.ds(j * 32, 32)],
                                        jnp.int32)
                                    va = plsc.bitcast(
                                        lax.shift_left(v, 16), jnp.float32)
                                    vb = plsc.bitcast(
                                        v & jnp.int32(-65536), jnp.float32)
                                    if k == 0:
                                        acc_a = wvec[0] * va
                                        acc_b = wvec[0] * vb
                                    else:
                                        acc_a += wvec[k] * va
                                        acc_b += wvec[k] * vb
                                plsc.addupdate(
                                    yl.at[pl.ds(rbase + j * 32, LN)], acc_a)
                                plsc.addupdate(
                                    yl.at[pl.ds(rbase + j * 32 + LN, LN)], acc_b)
                        else:
                            for j in range(C // LN):
                                acc = wvec[0] * rows[par, 0, ei, pl.ds(j * LN, LN)]
                                for k in range(1, 8):
                                    acc += wvec[k] * rows[par, k, ei,
                                                          pl.ds(j * LN, LN)]
                                plsc.addupdate(yl.at[pl.ds(rbase + j * LN, LN)],
                                               acc)
                        return cr2

                    lax.fori_loop(0, EB, edge, 0)

                    @pl.when(b + 2 < nb)
                    def _():
                        fire_rec(b0 + b + 2, par)

                    return cr

                lax.fori_loop(0, nb, blk, 0)
                pltpu.sync_copy(yl.at[pl.ds(0, CHN * C)],
                                y_h.at[pl.ds(n0 * C, CHN * C)])

            return carry

        lax.fori_loop(0, (NCHUNK + NW - 1) // NW, chunk_body, 0)

    return functools.partial(
        pl.kernel,
        out_type=jax.ShapeDtypeStruct((N * C,), jnp.float32),
        mesh=_mesh(),
        compiler_params=_SC_PARAMS,
        scratch_types=[
            pltpu.VMEM((144,), jnp.int32),
            pltpu.VMEM((C,), jnp.float32),
            pltpu.VMEM(((CHN + 1) * C,), jnp.float32),
            pltpu.VMEM((2, RPAD), jnp.int32),
            pltpu.VMEM((2, EB + LN), jnp.int32),
            pltpu.VMEM((2, 8, EB, C),
                       jnp.bfloat16 if packed else jnp.float32),
            pltpu.SemaphoreType.DMA((2,)),
            pltpu.SemaphoreType.DMA((2,)),
        ],
    )(body)


_PACKED = [False, False, False, False, False]
_CONVS = [_make_conv(c, p) for c, p in zip(COUT_PAD, _PACKED)]


def _perm(c):
    import numpy as _np
    out = []
    for g in range(c // 32):
        out.extend(range(g * 32, (g + 1) * 32, 2))
        out.extend(range(g * 32 + 1, (g + 1) * 32, 2))
    return _np.asarray(out, dtype=_np.int32)


def _mm(x, w2, relu, bf16_out):
    KK = w2.shape[0]
    CC = w2.shape[1]
    BN, BC = 400, 512
    odt = jnp.bfloat16 if bf16_out else jnp.float32

    def body(x_ref, w_ref, o_ref):
        xb = x_ref[:]
        if relu:
            xb = jnp.maximum(xb, 0.0)
        o_ref[:] = jnp.dot(
            xb, w_ref[:], preferred_element_type=jnp.float32
        ).astype(odt)

    return pl.pallas_call(
        body,
        grid=(N // BN, CC // BC),
        in_specs=[
            pl.BlockSpec((BN, KK), lambda i, j: (i, 0)),
            pl.BlockSpec((KK, BC), lambda i, j: (0, j)),
        ],
        out_specs=pl.BlockSpec((BN, BC), lambda i, j: (i, j)),
        out_shape=jax.ShapeDtypeStruct((N, CC), odt),
    )(x, w2)


def kernel(feats, pos, edge_index, W0, b0, W1, b1, W2, b2, W3, b3, W4, b4):
    src = edge_index[0]
    dst = edge_index[1]
    order = jnp.argsort(dst)
    src_s = src[order].astype(jnp.int32)
    dst_s = dst[order].astype(jnp.int32)
    rowptr = jnp.searchsorted(
        dst_s, jnp.arange(NCHUNK + 1, dtype=jnp.int32) * CHN
    ).astype(jnp.int32)
    rowptr = jnp.pad(rowptr, (0, 144 - (NCHUNK + 1)))
    srcp = jnp.pad(src_s, (0, EP - E))
    dstp = jnp.pad(dst_s, (0, EP - E))
    px = jnp.asarray(pos[:, 0])
    py = jnp.asarray(pos[:, 1])
    pz = jnp.asarray(pos[:, 2])

    rec = _geom(px, py, pz, srcp, dstp)

    params = [(W0, b0), (W1, b1), (W2, b2), (W3, b3), (W4, b4)]
    x = jnp.pad(feats, ((0, 0), (0, 8 - feats.shape[1])))
    y = None
    prev_perm = None
    for i, (W, b) in enumerate(params):
        cin = W.shape[1]
        kk = x.shape[1]
        cout = W.shape[2]
        cpad = COUT_PAD[i]
        w2 = jnp.transpose(W, (1, 0, 2))
        w2 = jnp.pad(w2, ((0, kk - cin), (0, 0), (0, cpad - cout)))
        w2 = w2.reshape(kk, K3 * cpad)
        if prev_perm is not None:
            w2 = w2[prev_perm, :]
        bp = jnp.pad(b, (0, cpad - cout))
        if _PACKED[i]:
            bp = bp[_perm(cpad)]
        T = _mm(x, w2, relu=(i > 0), bf16_out=_PACKED[i])
        T2 = T.reshape(N * K3, cpad)
        y = _CONVS[i](T2, rec, rowptr, bp)
        prev_perm = _perm(cpad) if _PACKED[i] else None
        if i < len(params) - 1:
            x = y.reshape(N, cpad)
    return y.reshape(N, COUT_PAD[-1])[:, : COUT[-1]]
